# baseline (speedup 1.0000x reference)
import functools

import jax
import jax.numpy as jnp
from jax.experimental import pallas as pl
from jax.experimental.pallas import tpu as pltpu

B = 4096
G = 19264
D = 64
NSTREAM = 4
BQ = 64
BM = NSTREAM * BQ


def _matmul_body(*refs):
    x_refs, emb_ref, out_ref = refs[:NSTREAM], refs[NSTREAM], refs[NSTREAM + 1]
    emb = emb_ref[...]
    for q in range(NSTREAM):
        out_ref[q * BQ:(q + 1) * BQ, :] = jax.lax.dot_general(
            x_refs[q][...], emb,
            dimension_numbers=(((1,), (0,)), ((), ())),
            preferred_element_type=jnp.float32,
        )


@functools.partial(jax.jit, static_argnames=())
def _feat(x, emb):
    grid = (B // BM,)
    x_specs = [
        pl.BlockSpec((BQ, G), functools.partial(lambda q, i: (NSTREAM * i + q, 0), q))
        for q in range(NSTREAM)
    ]
    return pl.pallas_call(
        _matmul_body,
        grid=grid,
        in_specs=x_specs + [pl.BlockSpec((G, D), lambda i: (0, 0))],
        out_specs=pl.BlockSpec((BM, D), lambda i: (i, 0)),
        out_shape=jax.ShapeDtypeStruct((B, D), jnp.float32),
    )(*([x] * NSTREAM), emb)


def kernel(x, emb):
    feat = _feat(x, emb)
    return (feat, emb)
